# baseline (speedup 1.0000x reference)
import functools

import jax
import jax.numpy as jnp
from jax import lax
from jax.experimental import pallas as pl
from jax.experimental.pallas import tpu as pltpu
from jax.experimental.pallas import tpu_sc as plsc

V = 30522
H = 768
P = 514
D = 50
U = 32
TAGV = 256
SUBV = 1024
B = 16
S = 512
N_TOK = B * S
PPAD = 640

DP = 56
KP = DP * U
NW = 32
TOK_PER_W = N_TOK // NW
WCH = 64
W_CHUNKS = TOK_PER_W // WCH
XT = 16
N_SUP = TOK_PER_W // XT


def _sc_xpath_body(tagidx_hbm, subidx_hbm, tagtab_hbm, subtab_hbm,
                   tag_out, sub_out, tidx_v, sidx_v, tval_v, sval_v, sem):
    wid = lax.axis_index("s") * 2 + lax.axis_index("c")

    pltpu.sync_copy(tagidx_hbm.at[wid], tidx_v)
    pltpu.sync_copy(subidx_hbm.at[wid], sidx_v)

    R = XT * DP

    def body(i, carry):
        r0 = i * R
        ct = pltpu.async_copy(tagtab_hbm.at[tidx_v.at[pl.ds(r0, R)]],
                              tval_v, sem)
        cs = pltpu.async_copy(subtab_hbm.at[sidx_v.at[pl.ds(r0, R)]],
                              sval_v, sem)
        o0 = wid * TOK_PER_W * DP + r0
        ct.wait()
        pltpu.sync_copy(tval_v, tag_out.at[pl.ds(o0, R)])
        cs.wait()
        pltpu.sync_copy(sval_v, sub_out.at[pl.ds(o0, R)])
        return carry

    lax.fori_loop(0, N_SUP, body, 0)


def _sc_word_body(ids_hbm, wemb_hbm, words_out, widx_v, wrows_v, sem):
    wid = lax.axis_index("s") * 2 + lax.axis_index("c")
    wbase = wid * TOK_PER_W
    pltpu.sync_copy(ids_hbm.at[wid], widx_v)
    for j in range(W_CHUNKS):
        pltpu.async_copy(wemb_hbm.at[widx_v.at[j]], wrows_v, sem).wait()
        pltpu.sync_copy(wrows_v, words_out.at[pl.ds(wbase + j * WCH, WCH)])


@functools.cache
def _make_sc_kernels():
    mesh = plsc.VectorSubcoreMesh(core_axis_name="c", subcore_axis_name="s")
    xpath = functools.partial(
        pl.kernel,
        mesh=mesh,
        out_type=[
            jax.ShapeDtypeStruct((N_TOK * DP, U), jnp.float32),
            jax.ShapeDtypeStruct((N_TOK * DP, U), jnp.float32),
        ],
        scratch_types=[
            pltpu.VMEM((TOK_PER_W * DP,), jnp.int32),
            pltpu.VMEM((TOK_PER_W * DP,), jnp.int32),
            pltpu.VMEM((XT * DP, U), jnp.float32),
            pltpu.VMEM((XT * DP, U), jnp.float32),
            pltpu.SemaphoreType.DMA,
        ],
        compiler_params=pltpu.CompilerParams(use_tc_tiling_on_sc=False),
    )(_sc_xpath_body)
    word = functools.partial(
        pl.kernel,
        mesh=mesh,
        out_type=jax.ShapeDtypeStruct((N_TOK, H), jnp.float32),
        scratch_types=[
            pltpu.VMEM((W_CHUNKS, WCH), jnp.int32),
            pltpu.VMEM((WCH, H), jnp.float32),
            pltpu.SemaphoreType.DMA,
        ],
        compiler_params=pltpu.CompilerParams(use_tc_tiling_on_sc=False),
    )(_sc_word_body)
    return xpath, word


def _tc1_body(tag_ref, sub_ref, w1_ref, b1_ref, w2_ref, b2_ref, xpe_ref):
    xp = (tag_ref[...] + sub_ref[...]).reshape(S, KP).astype(jnp.bfloat16)
    inner = jnp.dot(xp, w1_ref[...], preferred_element_type=jnp.float32)
    inner = jnp.maximum(inner + b1_ref[...], 0.0).astype(jnp.bfloat16)
    xpe = jnp.dot(inner, w2_ref[...], preferred_element_type=jnp.float32)
    xpe_ref[...] = xpe + b2_ref[...]


_tc1 = pl.pallas_call(
    _tc1_body,
    grid=(B,),
    in_specs=[
        pl.BlockSpec((S * KP // 128, 128), lambda i: (i, 0)),
        pl.BlockSpec((S * KP // 128, 128), lambda i: (i, 0)),
        pl.BlockSpec((KP, 4 * H), lambda i: (0, 0)),
        pl.BlockSpec((1, 4 * H), lambda i: (0, 0)),
        pl.BlockSpec((4 * H, H), lambda i: (0, 0)),
        pl.BlockSpec((1, H), lambda i: (0, 0)),
    ],
    out_specs=pl.BlockSpec((S, H), lambda i: (i, 0)),
    out_shape=jax.ShapeDtypeStruct((N_TOK, H), jnp.float32),
)


def _tc2_body(idsT_ref, words_ref, xpe_ref, pos_ref, tt_ref,
              g_ref, bta_ref, out_ref):
    maskf = (idsT_ref[...][0] != 0).astype(jnp.float32)
    row = lax.broadcasted_iota(jnp.int32, (S, S), 0)
    col = lax.broadcasted_iota(jnp.int32, (S, S), 1)
    tri = (col <= row).astype(jnp.float32)
    posid = jnp.dot(tri, maskf, preferred_element_type=jnp.float32) * maskf

    colp = lax.broadcasted_iota(jnp.int32, (S, PPAD), 1)
    oneh = (colp == posid.astype(jnp.int32)).astype(jnp.bfloat16)
    pos = jnp.dot(oneh, pos_ref[...], preferred_element_type=jnp.float32)

    words = words_ref[...].reshape(S, H)
    emb = words + pos + xpe_ref[...] + tt_ref[...]
    mu = jnp.mean(emb, axis=1, keepdims=True)
    var = jnp.mean(emb * emb, axis=1, keepdims=True) - mu * mu
    inv = lax.rsqrt(var + 1e-12)
    out_ref[...] = (emb - mu) * inv * g_ref[...] + bta_ref[...]


_tc2 = pl.pallas_call(
    _tc2_body,
    grid=(B,),
    in_specs=[
        pl.BlockSpec((1, S, 1), lambda i: (i, 0, 0)),
        pl.BlockSpec((S * H // 128, 128), lambda i: (i, 0)),
        pl.BlockSpec((S, H), lambda i: (i, 0)),
        pl.BlockSpec((PPAD, H), lambda i: (0, 0)),
        pl.BlockSpec((1, H), lambda i: (0, 0)),
        pl.BlockSpec((1, H), lambda i: (0, 0)),
        pl.BlockSpec((1, H), lambda i: (0, 0)),
    ],
    out_specs=pl.BlockSpec((S, H), lambda i: (i, 0)),
    out_shape=jax.ShapeDtypeStruct((N_TOK, H), jnp.float32),
)


def kernel(input_ids, xpath_tags_seq, xpath_subs_seq, word_emb, pos_emb,
           tok_type_emb, tag_tables, subs_tables, W_inner, b_inner, W_out,
           b_out, ln_gamma, ln_beta):
    doff = jnp.arange(D, dtype=jnp.int32)
    zpad = jnp.zeros((N_TOK, DP - D), jnp.int32)
    tag_idx = jnp.concatenate(
        [xpath_tags_seq.reshape(N_TOK, D) + doff * TAGV, zpad],
        axis=1).reshape(NW, -1)
    sub_idx = jnp.concatenate(
        [xpath_subs_seq.reshape(N_TOK, D) + doff * SUBV, zpad],
        axis=1).reshape(NW, -1)
    tagtab = tag_tables.reshape(D * TAGV, U)
    subtab = subs_tables.reshape(D * SUBV, U)

    sc_xpath, sc_word = _make_sc_kernels()
    tag_g, sub_g = sc_xpath(tag_idx, sub_idx, tagtab, subtab)
    words = sc_word(input_ids.reshape(NW, W_CHUNKS, WCH), word_emb)

    w1p = jnp.pad(W_inner.astype(jnp.bfloat16), ((0, KP - D * U), (0, 0)))
    xpe = _tc1(
        tag_g.reshape(N_TOK * KP // 128, 128),
        sub_g.reshape(N_TOK * KP // 128, 128),
        w1p,
        b_inner.reshape(1, 4 * H),
        W_out.astype(jnp.bfloat16),
        b_out.reshape(1, H),
    )

    pos_pad = jnp.zeros((PPAD, H), jnp.bfloat16).at[:P].set(
        pos_emb.astype(jnp.bfloat16))
    out = _tc2(
        input_ids.reshape(B, S, 1),
        words.reshape(N_TOK * H // 128, 128),
        xpe,
        pos_pad,
        tok_type_emb[0:1],
        ln_gamma.reshape(1, H),
        ln_beta.reshape(1, H),
    )
    return out.reshape(B, S, H)


# speedup vs baseline: 1.1120x; 1.1120x over previous
import functools

import jax
import jax.numpy as jnp
from jax import lax
from jax.experimental import pallas as pl
from jax.experimental.pallas import tpu as pltpu
from jax.experimental.pallas import tpu_sc as plsc

V = 30522
H = 768
P = 514
D = 50
U = 32
TAGV = 256
SUBV = 1024
B = 16
S = 512
N_TOK = B * S
PPAD = 640

DP = 56
KP = DP * U
NW = 32
TOK_PER_W = N_TOK // NW
WCH = 64
W_CHUNKS = TOK_PER_W // WCH
XT = 16
N_SUP = TOK_PER_W // XT


def _sc_xpath_body(tagidx_hbm, subidx_hbm, tagtab_hbm, subtab_hbm,
                   tag_out, sub_out, tidx_v, sidx_v, tval_v, sval_v, sem):
    wid = lax.axis_index("s") * 2 + lax.axis_index("c")

    pltpu.sync_copy(tagidx_hbm.at[wid], tidx_v)
    pltpu.sync_copy(subidx_hbm.at[wid], sidx_v)

    R = XT * DP

    def body(i, carry):
        r0 = i * R
        ct = pltpu.async_copy(tagtab_hbm.at[tidx_v.at[pl.ds(r0, R)]],
                              tval_v, sem)
        cs = pltpu.async_copy(subtab_hbm.at[sidx_v.at[pl.ds(r0, R)]],
                              sval_v, sem)
        o0 = wid * TOK_PER_W * DP + r0
        ct.wait()
        pltpu.sync_copy(tval_v, tag_out.at[pl.ds(o0, R)])
        cs.wait()
        pltpu.sync_copy(sval_v, sub_out.at[pl.ds(o0, R)])
        return carry

    lax.fori_loop(0, N_SUP, body, 0)


def _sc_word_body(ids_hbm, wemb_hbm, words_out, widx_v, wrows_v, sem):
    wid = lax.axis_index("s") * 2 + lax.axis_index("c")
    wbase = wid * TOK_PER_W
    pltpu.sync_copy(ids_hbm.at[wid], widx_v)
    for j in range(W_CHUNKS):
        pltpu.async_copy(wemb_hbm.at[widx_v.at[j]], wrows_v, sem).wait()
        pltpu.sync_copy(wrows_v, words_out.at[pl.ds(wbase + j * WCH, WCH)])


@functools.cache
def _make_sc_kernels():
    mesh = plsc.VectorSubcoreMesh(core_axis_name="c", subcore_axis_name="s")
    xpath = functools.partial(
        pl.kernel,
        mesh=mesh,
        out_type=[
            jax.ShapeDtypeStruct((N_TOK * DP, U), jnp.bfloat16),
            jax.ShapeDtypeStruct((N_TOK * DP, U), jnp.bfloat16),
        ],
        scratch_types=[
            pltpu.VMEM((TOK_PER_W * DP,), jnp.int32),
            pltpu.VMEM((TOK_PER_W * DP,), jnp.int32),
            pltpu.VMEM((XT * DP, U), jnp.bfloat16),
            pltpu.VMEM((XT * DP, U), jnp.bfloat16),
            pltpu.SemaphoreType.DMA,
        ],
        compiler_params=pltpu.CompilerParams(use_tc_tiling_on_sc=False),
    )(_sc_xpath_body)
    word = functools.partial(
        pl.kernel,
        mesh=mesh,
        out_type=jax.ShapeDtypeStruct((N_TOK, H), jnp.float32),
        scratch_types=[
            pltpu.VMEM((W_CHUNKS, WCH), jnp.int32),
            pltpu.VMEM((WCH, H), jnp.float32),
            pltpu.SemaphoreType.DMA,
        ],
        compiler_params=pltpu.CompilerParams(use_tc_tiling_on_sc=False),
    )(_sc_word_body)
    return xpath, word


def _tc1_body(tag_ref, sub_ref, w1_ref, b1_ref, w2_ref, b2_ref, xpe_ref):
    xp = (tag_ref[...] + sub_ref[...]).reshape(S, KP).astype(jnp.bfloat16)
    inner = jnp.dot(xp, w1_ref[...], preferred_element_type=jnp.float32)
    inner = jnp.maximum(inner + b1_ref[...], 0.0).astype(jnp.bfloat16)
    xpe = jnp.dot(inner, w2_ref[...], preferred_element_type=jnp.float32)
    xpe_ref[...] = xpe + b2_ref[...]


_tc1 = pl.pallas_call(
    _tc1_body,
    grid=(B,),
    in_specs=[
        pl.BlockSpec((S * KP // 128, 128), lambda i: (i, 0)),
        pl.BlockSpec((S * KP // 128, 128), lambda i: (i, 0)),
        pl.BlockSpec((KP, 4 * H), lambda i: (0, 0)),
        pl.BlockSpec((1, 4 * H), lambda i: (0, 0)),
        pl.BlockSpec((4 * H, H), lambda i: (0, 0)),
        pl.BlockSpec((1, H), lambda i: (0, 0)),
    ],
    out_specs=pl.BlockSpec((S, H), lambda i: (i, 0)),
    out_shape=jax.ShapeDtypeStruct((N_TOK, H), jnp.float32),
)


def _tc2_body(idsT_ref, words_ref, xpe_ref, pos_ref, tt_ref,
              g_ref, bta_ref, out_ref):
    maskf = (idsT_ref[...][0] != 0).astype(jnp.float32)
    row = lax.broadcasted_iota(jnp.int32, (S, S), 0)
    col = lax.broadcasted_iota(jnp.int32, (S, S), 1)
    tri = (col <= row).astype(jnp.float32)
    posid = jnp.dot(tri, maskf, preferred_element_type=jnp.float32) * maskf

    colp = lax.broadcasted_iota(jnp.int32, (S, PPAD), 1)
    oneh = (colp == posid.astype(jnp.int32)).astype(jnp.bfloat16)
    pos = jnp.dot(oneh, pos_ref[...], preferred_element_type=jnp.float32)

    words = words_ref[...].reshape(S, H)
    emb = words + pos + xpe_ref[...] + tt_ref[...]
    mu = jnp.mean(emb, axis=1, keepdims=True)
    var = jnp.mean(emb * emb, axis=1, keepdims=True) - mu * mu
    inv = lax.rsqrt(var + 1e-12)
    out_ref[...] = (emb - mu) * inv * g_ref[...] + bta_ref[...]


_tc2 = pl.pallas_call(
    _tc2_body,
    grid=(B,),
    in_specs=[
        pl.BlockSpec((1, S, 1), lambda i: (i, 0, 0)),
        pl.BlockSpec((S * H // 128, 128), lambda i: (i, 0)),
        pl.BlockSpec((S, H), lambda i: (i, 0)),
        pl.BlockSpec((PPAD, H), lambda i: (0, 0)),
        pl.BlockSpec((1, H), lambda i: (0, 0)),
        pl.BlockSpec((1, H), lambda i: (0, 0)),
        pl.BlockSpec((1, H), lambda i: (0, 0)),
    ],
    out_specs=pl.BlockSpec((S, H), lambda i: (i, 0)),
    out_shape=jax.ShapeDtypeStruct((N_TOK, H), jnp.float32),
)


def kernel(input_ids, xpath_tags_seq, xpath_subs_seq, word_emb, pos_emb,
           tok_type_emb, tag_tables, subs_tables, W_inner, b_inner, W_out,
           b_out, ln_gamma, ln_beta):
    doff = jnp.arange(D, dtype=jnp.int32)
    zpad = jnp.zeros((N_TOK, DP - D), jnp.int32)
    tag_idx = jnp.concatenate(
        [xpath_tags_seq.reshape(N_TOK, D) + doff * TAGV, zpad],
        axis=1).reshape(NW, -1)
    sub_idx = jnp.concatenate(
        [xpath_subs_seq.reshape(N_TOK, D) + doff * SUBV, zpad],
        axis=1).reshape(NW, -1)
    tagtab = tag_tables.reshape(D * TAGV, U).astype(jnp.bfloat16)
    subtab = subs_tables.reshape(D * SUBV, U).astype(jnp.bfloat16)

    sc_xpath, sc_word = _make_sc_kernels()
    tag_g, sub_g = sc_xpath(tag_idx, sub_idx, tagtab, subtab)
    words = sc_word(input_ids.reshape(NW, W_CHUNKS, WCH), word_emb)

    w1p = jnp.pad(W_inner.astype(jnp.bfloat16), ((0, KP - D * U), (0, 0)))
    xpe = _tc1(
        tag_g.reshape(N_TOK * KP // 128, 128),
        sub_g.reshape(N_TOK * KP // 128, 128),
        w1p,
        b_inner.reshape(1, 4 * H),
        W_out.astype(jnp.bfloat16),
        b_out.reshape(1, H),
    )

    pos_pad = jnp.zeros((PPAD, H), jnp.bfloat16).at[:P].set(
        pos_emb.astype(jnp.bfloat16))
    out = _tc2(
        input_ids.reshape(B, S, 1),
        words.reshape(N_TOK * H // 128, 128),
        xpe,
        pos_pad,
        tok_type_emb[0:1],
        ln_gamma.reshape(1, H),
        ln_beta.reshape(1, H),
    )
    return out.reshape(B, S, H)


# speedup vs baseline: 1.1127x; 1.0007x over previous
import functools

import jax
import jax.numpy as jnp
from jax import lax
from jax.experimental import pallas as pl
from jax.experimental.pallas import tpu as pltpu
from jax.experimental.pallas import tpu_sc as plsc

V = 30522
H = 768
P = 514
D = 50
U = 32
TAGV = 256
SUBV = 1024
B = 16
S = 512
N_TOK = B * S
PPAD = 640

DP = 56
KP = DP * U
NW = 32
TOK_PER_W = N_TOK // NW
WCH = 64
W_CHUNKS = TOK_PER_W // WCH
XT = 16
N_SUP = TOK_PER_W // XT


def _sc_xpath_body(tagidx_hbm, subidx_hbm, tagtab_hbm, subtab_hbm,
                   tag_out, sub_out, tidx_v, sidx_v, tval_v, sval_v, sem):
    wid = lax.axis_index("s") * 2 + lax.axis_index("c")

    pltpu.sync_copy(tagidx_hbm.at[wid], tidx_v)
    pltpu.sync_copy(subidx_hbm.at[wid], sidx_v)

    R = XT * DP
    obase = wid * TOK_PER_W * DP

    def gathers(i):
        p = (i % 2) * R
        ct = pltpu.async_copy(tagtab_hbm.at[tidx_v.at[pl.ds(i * R, R)]],
                              tval_v.at[pl.ds(p, R)], sem)
        cs = pltpu.async_copy(subtab_hbm.at[sidx_v.at[pl.ds(i * R, R)]],
                              sval_v.at[pl.ds(p, R)], sem)
        return ct, cs

    pend = gathers(0)
    for i in range(N_SUP):
        cur = pend
        if i + 1 < N_SUP:
            pend = gathers(i + 1)
        p = (i % 2) * R
        cur[0].wait()
        pltpu.sync_copy(tval_v.at[pl.ds(p, R)],
                        tag_out.at[pl.ds(obase + i * R, R)])
        cur[1].wait()
        pltpu.sync_copy(sval_v.at[pl.ds(p, R)],
                        sub_out.at[pl.ds(obase + i * R, R)])


def _sc_word_body(ids_hbm, wemb_hbm, words_out, widx_v, wrows_v, sem):
    wid = lax.axis_index("s") * 2 + lax.axis_index("c")
    wbase = wid * TOK_PER_W
    pltpu.sync_copy(ids_hbm.at[wid], widx_v)
    for j in range(W_CHUNKS):
        pltpu.async_copy(wemb_hbm.at[widx_v.at[j]], wrows_v, sem).wait()
        pltpu.sync_copy(wrows_v, words_out.at[pl.ds(wbase + j * WCH, WCH)])


@functools.cache
def _make_sc_kernels():
    mesh = plsc.VectorSubcoreMesh(core_axis_name="c", subcore_axis_name="s")
    xpath = functools.partial(
        pl.kernel,
        mesh=mesh,
        out_type=[
            jax.ShapeDtypeStruct((N_TOK * DP, U), jnp.bfloat16),
            jax.ShapeDtypeStruct((N_TOK * DP, U), jnp.bfloat16),
        ],
        scratch_types=[
            pltpu.VMEM((TOK_PER_W * DP,), jnp.int32),
            pltpu.VMEM((TOK_PER_W * DP,), jnp.int32),
            pltpu.VMEM((2 * XT * DP, U), jnp.bfloat16),
            pltpu.VMEM((2 * XT * DP, U), jnp.bfloat16),
            pltpu.SemaphoreType.DMA,
        ],
        compiler_params=pltpu.CompilerParams(use_tc_tiling_on_sc=False),
    )(_sc_xpath_body)
    word = functools.partial(
        pl.kernel,
        mesh=mesh,
        out_type=jax.ShapeDtypeStruct((N_TOK, H), jnp.float32),
        scratch_types=[
            pltpu.VMEM((W_CHUNKS, WCH), jnp.int32),
            pltpu.VMEM((WCH, H), jnp.float32),
            pltpu.SemaphoreType.DMA,
        ],
        compiler_params=pltpu.CompilerParams(use_tc_tiling_on_sc=False),
    )(_sc_word_body)
    return xpath, word


def _tc1_body(tag_ref, sub_ref, w1_ref, b1_ref, w2_ref, b2_ref, xpe_ref):
    xp = (tag_ref[...] + sub_ref[...]).reshape(S, KP).astype(jnp.bfloat16)
    inner = jnp.dot(xp, w1_ref[...], preferred_element_type=jnp.float32)
    inner = jnp.maximum(inner + b1_ref[...], 0.0).astype(jnp.bfloat16)
    xpe = jnp.dot(inner, w2_ref[...], preferred_element_type=jnp.float32)
    xpe_ref[...] = xpe + b2_ref[...]


_tc1 = pl.pallas_call(
    _tc1_body,
    grid=(B,),
    in_specs=[
        pl.BlockSpec((S * KP // 128, 128), lambda i: (i, 0)),
        pl.BlockSpec((S * KP // 128, 128), lambda i: (i, 0)),
        pl.BlockSpec((KP, 4 * H), lambda i: (0, 0)),
        pl.BlockSpec((1, 4 * H), lambda i: (0, 0)),
        pl.BlockSpec((4 * H, H), lambda i: (0, 0)),
        pl.BlockSpec((1, H), lambda i: (0, 0)),
    ],
    out_specs=pl.BlockSpec((S, H), lambda i: (i, 0)),
    out_shape=jax.ShapeDtypeStruct((N_TOK, H), jnp.float32),
)


def _tc2_body(idsT_ref, words_ref, xpe_ref, pos_ref, tt_ref,
              g_ref, bta_ref, out_ref):
    maskf = (idsT_ref[...][0] != 0).astype(jnp.float32)
    row = lax.broadcasted_iota(jnp.int32, (S, S), 0)
    col = lax.broadcasted_iota(jnp.int32, (S, S), 1)
    tri = (col <= row).astype(jnp.float32)
    posid = jnp.dot(tri, maskf, preferred_element_type=jnp.float32) * maskf

    colp = lax.broadcasted_iota(jnp.int32, (S, PPAD), 1)
    oneh = (colp == posid.astype(jnp.int32)).astype(jnp.bfloat16)
    pos = jnp.dot(oneh, pos_ref[...], preferred_element_type=jnp.float32)

    words = words_ref[...].reshape(S, H)
    emb = words + pos + xpe_ref[...] + tt_ref[...]
    mu = jnp.mean(emb, axis=1, keepdims=True)
    var = jnp.mean(emb * emb, axis=1, keepdims=True) - mu * mu
    inv = lax.rsqrt(var + 1e-12)
    out_ref[...] = (emb - mu) * inv * g_ref[...] + bta_ref[...]


_tc2 = pl.pallas_call(
    _tc2_body,
    grid=(B,),
    in_specs=[
        pl.BlockSpec((1, S, 1), lambda i: (i, 0, 0)),
        pl.BlockSpec((S * H // 128, 128), lambda i: (i, 0)),
        pl.BlockSpec((S, H), lambda i: (i, 0)),
        pl.BlockSpec((PPAD, H), lambda i: (0, 0)),
        pl.BlockSpec((1, H), lambda i: (0, 0)),
        pl.BlockSpec((1, H), lambda i: (0, 0)),
        pl.BlockSpec((1, H), lambda i: (0, 0)),
    ],
    out_specs=pl.BlockSpec((S, H), lambda i: (i, 0)),
    out_shape=jax.ShapeDtypeStruct((N_TOK, H), jnp.float32),
)


def kernel(input_ids, xpath_tags_seq, xpath_subs_seq, word_emb, pos_emb,
           tok_type_emb, tag_tables, subs_tables, W_inner, b_inner, W_out,
           b_out, ln_gamma, ln_beta):
    doff = jnp.arange(D, dtype=jnp.int32)
    zpad = jnp.zeros((N_TOK, DP - D), jnp.int32)
    tag_idx = jnp.concatenate(
        [xpath_tags_seq.reshape(N_TOK, D) + doff * TAGV, zpad],
        axis=1).reshape(NW, -1)
    sub_idx = jnp.concatenate(
        [xpath_subs_seq.reshape(N_TOK, D) + doff * SUBV, zpad],
        axis=1).reshape(NW, -1)
    tagtab = tag_tables.reshape(D * TAGV, U).astype(jnp.bfloat16)
    subtab = subs_tables.reshape(D * SUBV, U).astype(jnp.bfloat16)

    sc_xpath, sc_word = _make_sc_kernels()
    tag_g, sub_g = sc_xpath(tag_idx, sub_idx, tagtab, subtab)
    words = sc_word(input_ids.reshape(NW, W_CHUNKS, WCH), word_emb)

    w1p = jnp.pad(W_inner.astype(jnp.bfloat16), ((0, KP - D * U), (0, 0)))
    xpe = _tc1(
        tag_g.reshape(N_TOK * KP // 128, 128),
        sub_g.reshape(N_TOK * KP // 128, 128),
        w1p,
        b_inner.reshape(1, 4 * H),
        W_out.astype(jnp.bfloat16),
        b_out.reshape(1, H),
    )

    pos_pad = jnp.zeros((PPAD, H), jnp.bfloat16).at[:P].set(
        pos_emb.astype(jnp.bfloat16))
    out = _tc2(
        input_ids.reshape(B, S, 1),
        words.reshape(N_TOK * H // 128, 128),
        xpe,
        pos_pad,
        tok_type_emb[0:1],
        ln_gamma.reshape(1, H),
        ln_beta.reshape(1, H),
    )
    return out.reshape(B, S, H)
